# baseline (speedup 1.0000x reference)
import functools

import jax
import jax.numpy as jnp
from jax import lax
from jax.experimental import pallas as pl
from jax.experimental.pallas import tpu as pltpu
from jax.experimental.pallas import tpu_sc as plsc

VOCAB = 400000
EMBED_DIM = 300
BATCH = 16384

_ROW_BYTES = EMBED_DIM * 4
_GRAN = 16
_GPR = 20
_NGRAN = VOCAB * EMBED_DIM // _GRAN

_info = plsc.get_sparse_core_info()
_NC, _NS = _info.num_cores, _info.num_subcores
_NW = _NC * _NS
_B_PER_W = BATCH // _NW
_CHUNK = 32
_NCHUNK = _B_PER_W // _CHUNK
_IDX_PER_CHUNK = _CHUNK * _GPR
_NSTREAM = _IDX_PER_CHUNK // 128


def _gather_kernel(tview_hbm, gidx_hbm, qb_hbm, out_hbm,
                   idx_v, qb_v, staged_v, outb_v, gsem):
    wid = lax.axis_index("s") * _NC + lax.axis_index("c")
    iota16 = lax.iota(jnp.int32, 16)

    def do_chunk(c, carry):
        rowbase = wid * _B_PER_W + c * _CHUNK
        pltpu.sync_copy(gidx_hbm.at[wid, c], idx_v)
        pltpu.sync_copy(qb_hbm.at[wid, c], qb_v)
        for q in range(_NSTREAM):
            pltpu.async_copy(
                tview_hbm.at[idx_v.at[q]],
                staged_v.at[pl.ds(q * 128, 128)],
                gsem,
            ).wait()
        for g in range(_CHUNK // 16):
            qb16 = qb_v[pl.ds(g * 16, 16)]
            ri = g * 16 + iota16
            for k in range(EMBED_DIM):
                qk = qb16 + k
                v = plsc.load_gather(staged_v, [qk >> 4, qk & 15])
                plsc.store_scatter(
                    outb_v, [ri, jnp.full((16,), k, jnp.int32)], v
                )
        pltpu.sync_copy(outb_v, out_hbm.at[pl.ds(rowbase, _CHUNK)])
        return carry

    lax.fori_loop(0, _NCHUNK, do_chunk, 0)


@jax.jit
def _glove_gather(captions, glove_vectors):
    tview = glove_vectors.reshape(_NGRAN, _GRAN)
    cap = captions.reshape(_NW, _NCHUNK, _CHUNK)
    g0 = (cap * 75) // 4
    gidx = jnp.minimum(
        g0[..., None] + jnp.arange(_GPR, dtype=jnp.int32), _NGRAN - 1
    ).reshape(_NW, _NCHUNK, _NSTREAM, 128)
    qb = (_GPR * _GRAN) * jnp.arange(_CHUNK, dtype=jnp.int32)[None, None, :] \
        + (cap * EMBED_DIM) % _GRAN

    k = functools.partial(
        pl.kernel,
        out_type=jax.ShapeDtypeStruct((BATCH, EMBED_DIM), jnp.float32),
        mesh=plsc.VectorSubcoreMesh(core_axis_name="c", subcore_axis_name="s"),
        scratch_types=[
            pltpu.VMEM((_NSTREAM, 128), jnp.int32),
            pltpu.VMEM((_CHUNK,), jnp.int32),
            pltpu.VMEM((_IDX_PER_CHUNK, _GRAN), jnp.float32),
            pltpu.VMEM((_CHUNK, EMBED_DIM), jnp.float32),
            pltpu.SemaphoreType.DMA,
        ],
        compiler_params=pltpu.CompilerParams(
            use_tc_tiling_on_sc=False, needs_layout_passes=False
        ),
    )(_gather_kernel)
    return k(tview, gidx, qb)


def kernel(class_labels, captions, glove_vectors):
    return _glove_gather(captions, glove_vectors)


# speedup vs baseline: 1.7545x; 1.7545x over previous
import functools

import jax
import jax.numpy as jnp
from jax import lax
from jax.experimental import pallas as pl
from jax.experimental.pallas import tpu as pltpu
from jax.experimental.pallas import tpu_sc as plsc

VOCAB = 400000
EMBED_DIM = 300
BATCH = 16384

_info = plsc.get_sparse_core_info()
_NC, _NS = _info.num_cores, _info.num_subcores
_NW = _NC * _NS
_B_PER_W = BATCH // _NW
_CHUNK = 32
_NCHUNK = _B_PER_W // _CHUNK


def _gather_kernel(table_hbm, aux_hbm, idx_hbm, out_hbm,
                   idx_v, sA, sC, outb, gsemA, gsemC, osem):
    wid = lax.axis_index("s") * _NC + lax.axis_index("c")
    pltpu.sync_copy(idx_hbm.at[wid], idx_v)

    def fire(c, buf):
        iv = idx_v.at[c]
        pltpu.async_copy(table_hbm.at[iv, pl.ds(0, 256)], sA.at[buf], gsemA)
        pltpu.async_copy(aux_hbm.at[iv], sC.at[buf], gsemC)

    def wait_gathers(buf):
        pltpu.make_async_copy(table_hbm.at[idx_v.at[0], pl.ds(0, 256)],
                              sA.at[buf], gsemA).wait()
        pltpu.make_async_copy(aux_hbm.at[idx_v.at[0]], sC.at[buf],
                              gsemC).wait()

    def out_copy(c, buf):
        rowbase = wid * _B_PER_W + c * _CHUNK
        return pltpu.make_async_copy(
            outb.at[buf], out_hbm.at[pl.ds(rowbase, _CHUNK)], osem)

    def assemble(buf):
        for r in range(_CHUNK):
            for t in range(16):
                outb[buf, r, pl.ds(16 * t, 16)] = sA[buf, r, pl.ds(16 * t, 16)]
            outb[buf, r, pl.ds(256, 16)] = sC[buf, r, pl.ds(84, 16)]
            outb[buf, r, pl.ds(272, 16)] = sC[buf, r, pl.ds(100, 16)]
            outb[buf, r, pl.ds(284, 16)] = sC[buf, r, pl.ds(112, 16)]

    fire(0, 0)

    def do_pair(p, carry):
        c0 = 2 * p
        c1 = c0 + 1

        wait_gathers(0)
        fire(c1, 1)

        @pl.when(p >= 1)
        def _():
            out_copy(c0 - 2, 0).wait()

        assemble(0)
        out_copy(c0, 0).start()

        wait_gathers(1)

        @pl.when(p + 1 < _NCHUNK // 2)
        def _():
            fire(c1 + 1, 0)

        @pl.when(p >= 1)
        def _():
            out_copy(c1 - 2, 1).wait()

        assemble(1)
        out_copy(c1, 1).start()
        return carry

    lax.fori_loop(0, _NCHUNK // 2, do_pair, 0)
    out_copy(_NCHUNK - 2, 0).wait()
    out_copy(_NCHUNK - 1, 1).wait()


@jax.jit
def _glove_gather(captions, glove_vectors):
    aux = lax.slice(glove_vectors, (0, 172), (VOCAB, 300))
    idx = captions.reshape(_NW, _NCHUNK, _CHUNK)

    k = functools.partial(
        pl.kernel,
        out_type=jax.ShapeDtypeStruct((BATCH, EMBED_DIM), jnp.float32),
        mesh=plsc.VectorSubcoreMesh(core_axis_name="c", subcore_axis_name="s"),
        scratch_types=[
            pltpu.VMEM((_NCHUNK, _CHUNK), jnp.int32),
            pltpu.VMEM((2, _CHUNK, 256), jnp.float32),
            pltpu.VMEM((2, _CHUNK, 128), jnp.float32),
            pltpu.VMEM((2, _CHUNK, EMBED_DIM), jnp.float32),
            pltpu.SemaphoreType.DMA,
            pltpu.SemaphoreType.DMA,
            pltpu.SemaphoreType.DMA,
        ],
        compiler_params=pltpu.CompilerParams(
            use_tc_tiling_on_sc=True, needs_layout_passes=False
        ),
    )(_gather_kernel)
    return k(glove_vectors, aux, idx)


def kernel(class_labels, captions, glove_vectors):
    return _glove_gather(captions, glove_vectors)
